# baseline (speedup 1.0000x reference)
import jax
import jax.numpy as jnp
from jax import lax
from jax.experimental import pallas as pl
from jax.experimental.pallas import tpu as pltpu
from jax.experimental.pallas import tpu_sc as plsc

_N = 316
_N2 = _N * _N
_NC = 2
_NS = 16
_NT = 781
_CH = 3200
_TPW = 25
_HALO = 640
_HSLICE = _CH + 2 * _HALO
_JTOT = 13 * _CH

_mesh = plsc.VectorSubcoreMesh(
    core_axis_name="c", subcore_axis_name="s", num_cores=_NC, num_subcores=_NS
)


@pl.kernel(
    mesh=_mesh,
    out_type=jax.ShapeDtypeStruct((13, _NT, 2, 128), jnp.float32),
    scratch_types=[
        pltpu.VMEM((_HSLICE,), jnp.float32),
        pltpu.VMEM((_JTOT,), jnp.int32),
        pltpu.VMEM((13, _TPW, 2, 128), jnp.float32),
        pltpu.SemaphoreType.DMA,
        pltpu.SemaphoreType.DMA,
    ],
    compiler_params=pltpu.CompilerParams(needs_layout_passes=False),
)
def _sc_stencil(hs_hbm, idx_hbm, out_hbm, hs_v, idx_v, out_v, sem_in, sem_out):
    wid = lax.axis_index("s") * _NC + lax.axis_index("c")
    row0 = pl.multiple_of(jnp.minimum(wid * _CH, 96768), 128)
    it0 = jnp.minimum(wid * _TPW, _NT - _TPW)
    jstart = pl.multiple_of(jnp.minimum(row0 * 13, 13 * _N2 - _JTOT), 8)
    roff = row0 - jstart // 13
    lo = pl.multiple_of(jnp.clip(row0 - _HALO, 0, _N2 - _HSLICE), 8)
    idx_dma = pltpu.async_copy(idx_hbm.at[pl.ds(jstart, _JTOT)], idx_v, sem_in)
    pltpu.sync_copy(hs_hbm.at[pl.ds(lo, _HSLICE)], hs_v)

    soff = row0 - lo
    send = _HSLICE - 16 - soff
    iota13 = lax.iota(jnp.int32, 16) * 13

    @plsc.parallel_loop(0, 8 * _TPW, unroll=10)
    def _selfpass(s):
        it = lax.shift_right_logical(s, 3)
        bofs = (s & 7) * 16
        s0 = jnp.minimum(s * 16, send)
        v = hs_v[pl.ds(s0 + soff, 16)]
        for k in range(13):
            out_v[k, it, 0, pl.ds(bofs, 16)] = v

    del _selfpass
    idx_dma.wait()

    out_copies = []
    for k in range(13):
        bk = roff * 13 + k

        @plsc.parallel_loop(0, 8 * _TPW, unroll=10)
        def _kpass(s, _bk=bk, _k=k):
            it = lax.shift_right_logical(s, 3)
            bofs = (s & 7) * 16
            pos = jnp.minimum(s * 208 + _bk + iota13, _JTOT - 1)
            iv = plsc.load_gather(idx_v, [pos])
            g = plsc.load_gather(hs_v, [iv - lo])
            out_v[_k, it, 1, pl.ds(bofs, 16)] = g

        del _kpass
        out_copies.append(
            pltpu.async_copy(out_v.at[k], out_hbm.at[k, pl.ds(it0, _TPW)], sem_out)
        )
    for c in out_copies:
        c.wait()


def kernel(hs, index_list):
    x = _sc_stencil(hs, index_list)
    y = x.transpose(1, 3, 0, 2).reshape(_NT * 128, 13, 2)
    return y[:_N2]


# speedup vs baseline: 1.0395x; 1.0395x over previous
import jax
import jax.numpy as jnp
from jax import lax
from jax.experimental import pallas as pl
from jax.experimental.pallas import tpu as pltpu
from jax.experimental.pallas import tpu_sc as plsc

_N = 316
_N2 = _N * _N
_NC = 2
_NS = 16
_NT = 781
_CH = 3200
_TPW = 25
_HALO = 640
_HSLICE = _CH + 2 * _HALO
_JTOT = 13 * _CH

_mesh = plsc.VectorSubcoreMesh(
    core_axis_name="c", subcore_axis_name="s", num_cores=_NC, num_subcores=_NS
)


@pl.kernel(
    mesh=_mesh,
    out_type=jax.ShapeDtypeStruct((13, _NT, 2, 128), jnp.float32),
    scratch_types=[
        pltpu.VMEM((_HSLICE,), jnp.float32),
        pltpu.VMEM((_JTOT,), jnp.int32),
        pltpu.VMEM((13, _TPW, 2, 128), jnp.float32),
        pltpu.SemaphoreType.DMA,
        pltpu.SemaphoreType.DMA,
    ],
    compiler_params=pltpu.CompilerParams(needs_layout_passes=False),
)
def _sc_stencil(hs_hbm, idx_hbm, out_hbm, hs_v, idx_v, out_v, sem_in, sem_out):
    wid = lax.axis_index("s") * _NC + lax.axis_index("c")
    row0 = pl.multiple_of(jnp.minimum(wid * _CH, 96768), 128)
    it0 = jnp.minimum(wid * _TPW, _NT - _TPW)
    jstart = pl.multiple_of(jnp.minimum(row0 * 13, 13 * _N2 - _JTOT), 8)
    roff = row0 - jstart // 13
    lo = pl.multiple_of(jnp.clip(row0 - _HALO, 0, _N2 - _HSLICE), 8)
    idx_dma = pltpu.async_copy(idx_hbm.at[pl.ds(jstart, _JTOT)], idx_v, sem_in)
    pltpu.sync_copy(hs_hbm.at[pl.ds(lo, _HSLICE)], hs_v)

    soff = row0 - lo
    send = _HSLICE - 16 - soff
    iota13 = lax.iota(jnp.int32, 16) * 13

    @plsc.parallel_loop(0, 8 * _TPW, unroll=8)
    def _selfpass(s):
        it = lax.shift_right_logical(s, 3)
        bofs = (s & 7) * 16
        s0 = jnp.minimum(s * 16, send)
        v = hs_v[pl.ds(s0 + soff, 16)]
        for k in range(13):
            out_v[k, it, 0, pl.ds(bofs, 16)] = v

    del _selfpass
    idx_dma.wait()

    out_copies = []
    for k in range(13):
        bk = roff * 13 + k

        @plsc.parallel_loop(0, 8 * _TPW, unroll=8)
        def _kpass(s, _bk=bk, _k=k):
            it = lax.shift_right_logical(s, 3)
            bofs = (s & 7) * 16
            pos = jnp.minimum(s * 208 + _bk + iota13, _JTOT - 1)
            iv = plsc.load_gather(idx_v, [pos])
            g = plsc.load_gather(hs_v, [iv - lo])
            out_v[_k, it, 1, pl.ds(bofs, 16)] = g

        del _kpass
        out_copies.append(
            pltpu.async_copy(out_v.at[k], out_hbm.at[k, pl.ds(it0, _TPW)], sem_out)
        )
    for c in out_copies:
        c.wait()


def kernel(hs, index_list):
    x = _sc_stencil(hs, index_list)
    y = x.transpose(1, 3, 0, 2).reshape(_NT * 128, 13, 2)
    return y[:_N2]
